# baseline (speedup 1.0000x reference)
import functools
import jax
import jax.numpy as jnp
from jax import lax
from jax.experimental import pallas as pl
from jax.experimental.pallas import tpu as pltpu
from jax.experimental.pallas import tpu_sc as plsc

_E = 8
_EPAD = 128
_ROW_BLK = 128
_FFN_BLK = 512
_GATH_BLK = 1024


def _router_body(x_ref, gw_ref, w_ref, i_ref):
    x = x_ref[...]
    gw = gw_ref[...]
    logits = lax.dot_general(
        x, gw, (((1,), (1,)), ((), ())), preferred_element_type=jnp.float32)
    col = lax.broadcasted_iota(jnp.int32, logits.shape, 1)
    valid = col < _E
    l = jnp.where(valid, logits, -1e30)
    m = jnp.max(l, axis=1, keepdims=True)
    p = jnp.exp(l - m)
    p = jnp.where(valid, p, 0.0)
    probs = p / jnp.sum(p, axis=1, keepdims=True)
    w0 = jnp.max(probs, axis=1, keepdims=True)
    i0 = jnp.min(jnp.where(probs == w0, col, _E), axis=1, keepdims=True)
    probs1 = jnp.where(col == i0, -1.0, probs)
    w1 = jnp.max(probs1, axis=1, keepdims=True)
    i1 = jnp.min(jnp.where(probs1 == w1, col, _E), axis=1, keepdims=True)
    s = w0 + w1
    w_ref[...] = jnp.where(col == 0, w0 / s, jnp.where(col == 1, w1 / s, 0.0))
    i_ref[...] = jnp.where(col == 0, i0, jnp.where(col == 1, i1, 0))


def _run_router(x, gate_w):
    T, D = x.shape
    gwp = jnp.zeros((_EPAD, D), x.dtype).at[:_E].set(gate_w)
    return pl.pallas_call(
        _router_body,
        out_shape=(jax.ShapeDtypeStruct((T, _EPAD), jnp.float32),
                   jax.ShapeDtypeStruct((T, _EPAD), jnp.int32)),
    )(x, gwp)


def _run_gather_tc(x, tok2d, S_max):
    T, D = x.shape
    NB = S_max // _GATH_BLK

    def body(x_ref, tok_ref, xg_ref, xbf_ref):
        s = pl.program_id(0)

        @pl.when(s == 0)
        def _():
            xbf_ref[...] = x_ref[...].astype(jnp.bfloat16)

        tok = tok_ref[:, 0:1]
        oh = (lax.broadcasted_iota(jnp.int32, (_GATH_BLK, T), 1) == tok
              ).astype(jnp.bfloat16)
        xg_ref[...] = lax.dot_general(
            oh, xbf_ref[...], (((1,), (0,)), ((), ())),
            preferred_element_type=jnp.float32).astype(jnp.bfloat16)

    return pl.pallas_call(
        body,
        grid=(NB,),
        in_specs=[
            pl.BlockSpec((T, D), lambda s: (0, 0)),
            pl.BlockSpec((_GATH_BLK, _E), lambda s: (s, 0)),
        ],
        out_specs=pl.BlockSpec((_GATH_BLK, D), lambda s: (s, 0)),
        scratch_shapes=[pltpu.VMEM((T, D), jnp.bfloat16)],
        out_shape=jax.ShapeDtypeStruct((S_max, D), jnp.bfloat16),
        compiler_params=pltpu.CompilerParams(
            dimension_semantics=("arbitrary",)),
    )(x, tok2d)


def _gmm_body(pref_ref, xg_hbm, w1_ref, w3_ref, w2_ref, sw_ref, ys_hbm,
              acc_ref, xg_ref, sem_x, sem_o):
    j = pl.program_id(0)
    e = pl.program_id(1)
    nj = pl.num_programs(0)
    nb = pref_ref[0, e]
    pb = pref_ref[1, e]

    @pl.when((j == 0) & (e == 0))
    def _():
        cp = pltpu.make_async_copy(xg_hbm, xg_ref, sem_x)
        cp.start()
        cp.wait()

    w1 = w1_ref[0].astype(jnp.bfloat16)
    w3 = w3_ref[0].astype(jnp.bfloat16)
    w2 = w2_ref[0].astype(jnp.bfloat16)

    def compute_mb(mb, carry):
        r0 = (pb + mb) * _ROW_BLK
        sl = pl.ds(r0, _ROW_BLK)
        xg = xg_ref[sl, :]
        a = lax.dot_general(xg, w1, (((1,), (1,)), ((), ())),
                            preferred_element_type=jnp.float32)
        b = lax.dot_general(xg, w3, (((1,), (1,)), ((), ())),
                            preferred_element_type=jnp.float32)
        h = (a * (1.0 / (1.0 + jnp.exp(-a))) * b).astype(jnp.bfloat16)
        part = lax.dot_general(h, w2, (((1,), (1,)), ((), ())),
                               preferred_element_type=jnp.float32)

        @pl.when(j == 0)
        def _():
            acc_ref[sl, :] = part

        @pl.when((j > 0) & (j < nj - 1))
        def _():
            acc_ref[sl, :] += part

        @pl.when(j == nj - 1)
        def _():
            acc_ref[sl, :] = ((acc_ref[sl, :] + part)
                              * sw_ref[sl, 0:1].astype(jnp.float32))

        return carry

    lax.fori_loop(0, nb, compute_mb, 0)

    @pl.when(j == nj - 1)
    def _():
        def fire(mb, carry):
            r0 = (pb + mb) * _ROW_BLK
            pltpu.make_async_copy(
                acc_ref.at[pl.ds(r0, _ROW_BLK)],
                ys_hbm.at[pl.ds(r0, _ROW_BLK)], sem_o).start()
            return carry

        lax.fori_loop(0, nb, fire, 0)

        def drain(mb, carry):
            pltpu.make_async_copy(
                acc_ref.at[pl.ds(0, _ROW_BLK)],
                ys_hbm.at[pl.ds(0, _ROW_BLK)], sem_o).wait()
            return carry

        lax.fori_loop(0, nb, drain, 0)


def _run_gmm(pref, xg, w1, w3, w2, sw2d):
    S_max, D = xg.shape
    FFN = w1.shape[1]
    J = FFN // _FFN_BLK
    grid_spec = pltpu.PrefetchScalarGridSpec(
        num_scalar_prefetch=1,
        grid=(J, _E),
        in_specs=[
            pl.BlockSpec(memory_space=pl.ANY),
            pl.BlockSpec((1, _FFN_BLK, D), lambda j, e, pref: (e, j, 0)),
            pl.BlockSpec((1, _FFN_BLK, D), lambda j, e, pref: (e, j, 0)),
            pl.BlockSpec((1, D, _FFN_BLK), lambda j, e, pref: (e, 0, j)),
            pl.BlockSpec((S_max, _E), lambda j, e, pref: (0, 0)),
        ],
        out_specs=pl.BlockSpec(memory_space=pl.ANY),
        scratch_shapes=[
            pltpu.VMEM((S_max, D), jnp.float32),
            pltpu.VMEM((S_max, D), jnp.bfloat16),
            pltpu.SemaphoreType.DMA,
            pltpu.SemaphoreType.DMA,
        ],
    )
    return pl.pallas_call(
        _gmm_body,
        grid_spec=grid_spec,
        out_shape=jax.ShapeDtypeStruct((S_max, D), jnp.float32),
        compiler_params=pltpu.CompilerParams(
            dimension_semantics=("arbitrary", "arbitrary")),
    )(pref, xg, w1, w3, w2, sw2d)


def _run_sc_combine(ys, inv0, inv1):
    T = inv0.shape[0]
    D = ys.shape[1]
    info = plsc.get_sparse_core_info()
    nw = info.num_cores * info.num_subcores
    tok_pw = T // nw
    ch = 32
    n_ch = tok_pw // ch
    vecs_per_row = D // 16
    mesh = plsc.VectorSubcoreMesh(core_axis_name="c", subcore_axis_name="s")

    @functools.partial(
        pl.kernel,
        out_type=jax.ShapeDtypeStruct((T, D), jnp.float32),
        mesh=mesh,
        scratch_types=[
            pltpu.VMEM((ch,), jnp.int32),
            pltpu.VMEM((ch,), jnp.int32),
            pltpu.VMEM((ch, D), jnp.float32),
            pltpu.VMEM((ch, D), jnp.float32),
            pltpu.SemaphoreType.DMA,
            pltpu.SemaphoreType.DMA,
        ],
    )
    def k(ys_hbm, i0_hbm, i1_hbm, out_hbm, i0_v, i1_v, r0, r1, s0, s1):
        wid = lax.axis_index("s") * info.num_cores + lax.axis_index("c")
        base = wid * tok_pw
        for c in range(n_ch):
            off = base + c * ch
            pltpu.sync_copy(i0_hbm.at[pl.ds(off, ch)], i0_v)
            pltpu.sync_copy(i1_hbm.at[pl.ds(off, ch)], i1_v)
            cp0 = pltpu.async_copy(ys_hbm.at[i0_v], r0, s0)
            cp1 = pltpu.async_copy(ys_hbm.at[i1_v], r1, s1)
            cp0.wait()
            cp1.wait()

            def row_add(r, carry):
                for v in range(vecs_per_row):
                    sl = pl.ds(v * 16, 16)
                    r0[r, sl] = r0[r, sl] + r1[r, sl]
                return carry

            lax.fori_loop(0, ch, row_add, 0)
            pltpu.sync_copy(r0, out_hbm.at[pl.ds(off, ch)])

    return k(ys, inv0, inv1)


def kernel(hidden_states, gate_w, w1, w3, w2):
    B, S, D = hidden_states.shape
    T = B * S
    x = hidden_states.reshape(T, D)
    S_max = 2 * T + _E * _ROW_BLK
    NB = S_max // _ROW_BLK

    wpad, ipad = _run_router(x, gate_w)
    tw = wpad[:, :2]
    ti = ipad[:, :2]

    e_flat = jnp.concatenate([ti[:, 0], ti[:, 1]])
    w_flat = jnp.concatenate([tw[:, 0], tw[:, 1]])
    onehot = (e_flat[:, None] == jnp.arange(_E, dtype=jnp.int32)[None, :]
              ).astype(jnp.int32)
    ranks_all = jnp.cumsum(onehot, axis=0) - onehot
    rank = jnp.take_along_axis(ranks_all, e_flat[:, None], axis=1)[:, 0]
    counts = jnp.sum(onehot, axis=0)
    pcounts = ((counts + _ROW_BLK - 1) // _ROW_BLK) * _ROW_BLK
    pstart = jnp.concatenate(
        [jnp.zeros((1,), jnp.int32), jnp.cumsum(pcounts)[:-1].astype(jnp.int32)])
    pos = pstart[e_flat] + rank

    tok2 = jnp.arange(2 * T, dtype=jnp.int32) % T
    tok2d = jnp.zeros((S_max, _E), jnp.int32).at[pos].set(
        jnp.broadcast_to(tok2[:, None], (2 * T, _E)))
    sw2d = jnp.zeros((S_max, _E), jnp.bfloat16).at[pos].set(
        jnp.broadcast_to(w_flat[:, None].astype(jnp.bfloat16), (2 * T, _E)))

    pref = jnp.stack([pcounts // _ROW_BLK, pstart // _ROW_BLK]
                     ).astype(jnp.int32)
    inv0 = pos[:T]
    inv1 = pos[T:]

    xg = _run_gather_tc(x, tok2d, S_max)
    ys = _run_gmm(pref, xg, w1, w3, w2, sw2d)
    out = _run_sc_combine(ys, inv0, inv1)
    return out.reshape(B, S, D)


# speedup vs baseline: 1.5058x; 1.5058x over previous
import functools
import jax
import jax.numpy as jnp
from jax import lax
from jax.experimental import pallas as pl
from jax.experimental.pallas import tpu as pltpu
from jax.experimental.pallas import tpu_sc as plsc

_E = 8
_EPAD = 128
_ROW_BLK = 256
_FFN_BLK = 512
_GATH_BLK = 1024


def _router_body(x_ref, gw_ref, w_ref, i_ref):
    x = x_ref[...]
    gw = gw_ref[...]
    logits = lax.dot_general(
        x, gw, (((1,), (1,)), ((), ())), preferred_element_type=jnp.float32)
    col = lax.broadcasted_iota(jnp.int32, logits.shape, 1)
    valid = col < _E
    l = jnp.where(valid, logits, -1e30)
    m = jnp.max(l, axis=1, keepdims=True)
    p = jnp.exp(l - m)
    p = jnp.where(valid, p, 0.0)
    probs = p / jnp.sum(p, axis=1, keepdims=True)
    w0 = jnp.max(probs, axis=1, keepdims=True)
    i0 = jnp.min(jnp.where(probs == w0, col, _E), axis=1, keepdims=True)
    probs1 = jnp.where(col == i0, -1.0, probs)
    w1 = jnp.max(probs1, axis=1, keepdims=True)
    i1 = jnp.min(jnp.where(probs1 == w1, col, _E), axis=1, keepdims=True)
    s = w0 + w1
    w_ref[...] = jnp.where(col == 0, w0 / s, jnp.where(col == 1, w1 / s, 0.0))
    i_ref[...] = jnp.where(col == 0, i0, jnp.where(col == 1, i1, 0))


def _run_router(x, gate_w):
    T, D = x.shape
    gwp = jnp.zeros((_EPAD, D), x.dtype).at[:_E].set(gate_w)
    return pl.pallas_call(
        _router_body,
        out_shape=(jax.ShapeDtypeStruct((T, _EPAD), jnp.float32),
                   jax.ShapeDtypeStruct((T, _EPAD), jnp.int32)),
    )(x, gwp)


def _run_gather_tc(x, tok2d, S_max):
    T, D = x.shape
    NB = S_max // _GATH_BLK

    def body(x_ref, tok_ref, xg_ref, xbf_ref):
        s = pl.program_id(0)

        @pl.when(s == 0)
        def _():
            xbf_ref[...] = x_ref[...].astype(jnp.bfloat16)

        tok = tok_ref[:, 0:1]
        oh = (lax.broadcasted_iota(jnp.int32, (_GATH_BLK, T), 1) == tok
              ).astype(jnp.bfloat16)
        xg_ref[...] = lax.dot_general(
            oh, xbf_ref[...], (((1,), (0,)), ((), ())),
            preferred_element_type=jnp.float32).astype(jnp.bfloat16)

    return pl.pallas_call(
        body,
        grid=(NB,),
        in_specs=[
            pl.BlockSpec((T, D), lambda s: (0, 0)),
            pl.BlockSpec((_GATH_BLK, _E), lambda s: (s, 0)),
        ],
        out_specs=pl.BlockSpec((_GATH_BLK, D), lambda s: (s, 0)),
        scratch_shapes=[pltpu.VMEM((T, D), jnp.bfloat16)],
        out_shape=jax.ShapeDtypeStruct((S_max, D), jnp.bfloat16),
        compiler_params=pltpu.CompilerParams(
            dimension_semantics=("arbitrary",)),
    )(x, tok2d)


def _gmm_body(pref_ref, xg_hbm, w1_ref, w3_ref, w2_ref, sw_ref, ys_hbm,
              acc_ref, xg_ref, sem_x, sem_o):
    j = pl.program_id(0)
    e = pl.program_id(1)
    nj = pl.num_programs(0)
    nb = pref_ref[0, e]
    pb = pref_ref[1, e]

    @pl.when((j == 0) & (e == 0))
    def _():
        cp = pltpu.make_async_copy(xg_hbm, xg_ref, sem_x)
        cp.start()
        cp.wait()

    w1 = w1_ref[0].astype(jnp.bfloat16)
    w3 = w3_ref[0].astype(jnp.bfloat16)
    w2 = w2_ref[0].astype(jnp.bfloat16)

    def compute_mb(mb, carry):
        r0 = (pb + mb) * _ROW_BLK
        sl = pl.ds(r0, _ROW_BLK)
        xg = xg_ref[sl, :]
        a = lax.dot_general(xg, w1, (((1,), (1,)), ((), ())),
                            preferred_element_type=jnp.float32)
        b = lax.dot_general(xg, w3, (((1,), (1,)), ((), ())),
                            preferred_element_type=jnp.float32)
        h = (a * (1.0 / (1.0 + jnp.exp(-a))) * b).astype(jnp.bfloat16)
        part = lax.dot_general(h, w2, (((1,), (1,)), ((), ())),
                               preferred_element_type=jnp.float32)

        @pl.when(j == 0)
        def _():
            acc_ref[sl, :] = part

        @pl.when((j > 0) & (j < nj - 1))
        def _():
            acc_ref[sl, :] += part

        @pl.when(j == nj - 1)
        def _():
            acc_ref[sl, :] = ((acc_ref[sl, :] + part)
                              * sw_ref[sl, 0:1].astype(jnp.float32))

        return carry

    lax.fori_loop(0, nb, compute_mb, 0)

    @pl.when(j == nj - 1)
    def _():
        def fire(mb, carry):
            r0 = (pb + mb) * _ROW_BLK
            pltpu.make_async_copy(
                acc_ref.at[pl.ds(r0, _ROW_BLK)],
                ys_hbm.at[pl.ds(r0, _ROW_BLK)], sem_o).start()
            return carry

        lax.fori_loop(0, nb, fire, 0)

        def drain(mb, carry):
            pltpu.make_async_copy(
                acc_ref.at[pl.ds(0, _ROW_BLK)],
                ys_hbm.at[pl.ds(0, _ROW_BLK)], sem_o).wait()
            return carry

        lax.fori_loop(0, nb, drain, 0)


def _run_gmm(pref, xg, w1, w3, w2, sw2d):
    S_max, D = xg.shape
    FFN = w1.shape[1]
    J = FFN // _FFN_BLK
    grid_spec = pltpu.PrefetchScalarGridSpec(
        num_scalar_prefetch=1,
        grid=(J, _E),
        in_specs=[
            pl.BlockSpec(memory_space=pl.ANY),
            pl.BlockSpec((1, _FFN_BLK, D), lambda j, e, pref: (e, j, 0)),
            pl.BlockSpec((1, _FFN_BLK, D), lambda j, e, pref: (e, j, 0)),
            pl.BlockSpec((1, D, _FFN_BLK), lambda j, e, pref: (e, 0, j)),
            pl.BlockSpec((S_max, _E), lambda j, e, pref: (0, 0)),
        ],
        out_specs=pl.BlockSpec(memory_space=pl.ANY),
        scratch_shapes=[
            pltpu.VMEM((S_max, D), jnp.float32),
            pltpu.VMEM((S_max, D), jnp.bfloat16),
            pltpu.SemaphoreType.DMA,
            pltpu.SemaphoreType.DMA,
        ],
    )
    return pl.pallas_call(
        _gmm_body,
        grid_spec=grid_spec,
        out_shape=jax.ShapeDtypeStruct((S_max, D), jnp.float32),
        compiler_params=pltpu.CompilerParams(
            dimension_semantics=("arbitrary", "arbitrary")),
    )(pref, xg, w1, w3, w2, sw2d)


def _run_sc_combine(ys, inv0, inv1):
    T = inv0.shape[0]
    D = ys.shape[1]
    info = plsc.get_sparse_core_info()
    nw = info.num_cores * info.num_subcores
    tok_pw = T // nw
    ch = 32
    n_ch = tok_pw // ch
    vecs_per_row = D // 16
    mesh = plsc.VectorSubcoreMesh(core_axis_name="c", subcore_axis_name="s")

    @functools.partial(
        pl.kernel,
        out_type=jax.ShapeDtypeStruct((T, D), jnp.float32),
        mesh=mesh,
        scratch_types=[
            pltpu.VMEM((ch,), jnp.int32),
            pltpu.VMEM((ch,), jnp.int32),
            pltpu.VMEM((ch, D), jnp.float32),
            pltpu.VMEM((ch, D), jnp.float32),
            pltpu.SemaphoreType.DMA,
            pltpu.SemaphoreType.DMA,
        ],
    )
    def k(ys_hbm, i0_hbm, i1_hbm, out_hbm, i0_v, i1_v, r0, r1, s0, s1):
        wid = lax.axis_index("s") * info.num_cores + lax.axis_index("c")
        base = wid * tok_pw
        for c in range(n_ch):
            off = base + c * ch
            pltpu.sync_copy(i0_hbm.at[pl.ds(off, ch)], i0_v)
            pltpu.sync_copy(i1_hbm.at[pl.ds(off, ch)], i1_v)
            cp0 = pltpu.async_copy(ys_hbm.at[i0_v], r0, s0)
            cp1 = pltpu.async_copy(ys_hbm.at[i1_v], r1, s1)
            cp0.wait()
            cp1.wait()

            def row_add(r, carry):
                for v in range(vecs_per_row):
                    sl = pl.ds(v * 16, 16)
                    r0[r, sl] = r0[r, sl] + r1[r, sl]
                return carry

            lax.fori_loop(0, ch, row_add, 0)
            pltpu.sync_copy(r0, out_hbm.at[pl.ds(off, ch)])

    return k(ys, inv0, inv1)


def kernel(hidden_states, gate_w, w1, w3, w2):
    B, S, D = hidden_states.shape
    T = B * S
    x = hidden_states.reshape(T, D)
    S_max = 2 * T + _E * _ROW_BLK
    NB = S_max // _ROW_BLK

    wpad, ipad = _run_router(x, gate_w)
    tw = wpad[:, :2]
    ti = ipad[:, :2]

    e_flat = jnp.concatenate([ti[:, 0], ti[:, 1]])
    w_flat = jnp.concatenate([tw[:, 0], tw[:, 1]])
    onehot = (e_flat[:, None] == jnp.arange(_E, dtype=jnp.int32)[None, :]
              ).astype(jnp.int32)
    ranks_all = jnp.cumsum(onehot, axis=0) - onehot
    rank = jnp.take_along_axis(ranks_all, e_flat[:, None], axis=1)[:, 0]
    counts = jnp.sum(onehot, axis=0)
    pcounts = ((counts + _ROW_BLK - 1) // _ROW_BLK) * _ROW_BLK
    pstart = jnp.concatenate(
        [jnp.zeros((1,), jnp.int32), jnp.cumsum(pcounts)[:-1].astype(jnp.int32)])
    pos = pstart[e_flat] + rank

    tok2 = jnp.arange(2 * T, dtype=jnp.int32) % T
    tok2d = jnp.zeros((S_max, _E), jnp.int32).at[pos].set(
        jnp.broadcast_to(tok2[:, None], (2 * T, _E)))
    sw2d = jnp.zeros((S_max, _E), jnp.bfloat16).at[pos].set(
        jnp.broadcast_to(w_flat[:, None].astype(jnp.bfloat16), (2 * T, _E)))

    pref = jnp.stack([pcounts // _ROW_BLK, pstart // _ROW_BLK]
                     ).astype(jnp.int32)
    inv0 = pos[:T]
    inv1 = pos[T:]

    xg = _run_gather_tc(x, tok2d, S_max)
    ys = _run_gmm(pref, xg, w1, w3, w2, sw2d)
    out = _run_sc_combine(ys, inv0, inv1)
    return out.reshape(B, S, D)


# speedup vs baseline: 1.5085x; 1.0018x over previous
import functools
import jax
import jax.numpy as jnp
from jax import lax
from jax.experimental import pallas as pl
from jax.experimental.pallas import tpu as pltpu
from jax.experimental.pallas import tpu_sc as plsc

_E = 8
_EPAD = 128
_ROW_BLK = 256
_FFN_BLK = 512
_GATH_BLK = 1024


def _router_body(x_ref, gw_ref, w_ref, i_ref):
    x = x_ref[...]
    gw = gw_ref[...]
    logits = lax.dot_general(
        x, gw, (((1,), (1,)), ((), ())), preferred_element_type=jnp.float32)
    col = lax.broadcasted_iota(jnp.int32, logits.shape, 1)
    valid = col < _E
    l = jnp.where(valid, logits, -1e30)
    m = jnp.max(l, axis=1, keepdims=True)
    p = jnp.exp(l - m)
    p = jnp.where(valid, p, 0.0)
    probs = p / jnp.sum(p, axis=1, keepdims=True)
    w0 = jnp.max(probs, axis=1, keepdims=True)
    i0 = jnp.min(jnp.where(probs == w0, col, _E), axis=1, keepdims=True)
    probs1 = jnp.where(col == i0, -1.0, probs)
    w1 = jnp.max(probs1, axis=1, keepdims=True)
    i1 = jnp.min(jnp.where(probs1 == w1, col, _E), axis=1, keepdims=True)
    s = w0 + w1
    w_ref[...] = jnp.where(col == 0, w0 / s, jnp.where(col == 1, w1 / s, 0.0))
    i_ref[...] = jnp.where(col == 0, i0, jnp.where(col == 1, i1, 0))


def _run_router(x, gate_w):
    T, D = x.shape
    gwp = jnp.zeros((_EPAD, D), x.dtype).at[:_E].set(gate_w)
    return pl.pallas_call(
        _router_body,
        out_shape=(jax.ShapeDtypeStruct((T, _EPAD), jnp.float32),
                   jax.ShapeDtypeStruct((T, _EPAD), jnp.int32)),
    )(x, gwp)


def _run_gather_tc(x, tok2d, S_max):
    T, D = x.shape
    NB = S_max // _GATH_BLK

    def body(x_ref, tok_ref, xg_ref, xbf_ref):
        s = pl.program_id(0)

        @pl.when(s == 0)
        def _():
            xbf_ref[...] = x_ref[...].astype(jnp.bfloat16)

        tok = tok_ref[:, 0:1]
        oh = (lax.broadcasted_iota(jnp.int32, (_GATH_BLK, T), 1) == tok
              ).astype(jnp.bfloat16)
        xg_ref[...] = lax.dot_general(
            oh, xbf_ref[...], (((1,), (0,)), ((), ())),
            preferred_element_type=jnp.float32).astype(jnp.bfloat16)

    return pl.pallas_call(
        body,
        grid=(NB,),
        in_specs=[
            pl.BlockSpec((T, D), lambda s: (0, 0)),
            pl.BlockSpec((_GATH_BLK, _E), lambda s: (s, 0)),
        ],
        out_specs=pl.BlockSpec((_GATH_BLK, D), lambda s: (s, 0)),
        scratch_shapes=[pltpu.VMEM((T, D), jnp.bfloat16)],
        out_shape=jax.ShapeDtypeStruct((S_max, D), jnp.bfloat16),
        compiler_params=pltpu.CompilerParams(
            dimension_semantics=("arbitrary",)),
    )(x, tok2d)


def _gmm_body(pref_ref, xg_hbm, w1_ref, w3_ref, w2_ref, sw_ref, ys_hbm,
              acc_ref, xg_ref, sem_x, sem_o):
    j = pl.program_id(0)
    e = pl.program_id(1)
    nj = pl.num_programs(0)
    nb = pref_ref[0, e]
    pb = pref_ref[1, e]

    @pl.when((j == 0) & (e == 0))
    def _():
        cp = pltpu.make_async_copy(xg_hbm, xg_ref, sem_x)
        cp.start()
        cp.wait()

    w1 = w1_ref[0].astype(jnp.bfloat16)
    w3 = w3_ref[0].astype(jnp.bfloat16)
    w2 = w2_ref[0].astype(jnp.bfloat16)

    def compute_mb(mb, carry):
        r0 = (pb + mb) * _ROW_BLK
        sl = pl.ds(r0, _ROW_BLK)
        xg = xg_ref[sl, :]
        a = lax.dot_general(xg, w1, (((1,), (1,)), ((), ())),
                            preferred_element_type=jnp.float32)
        b = lax.dot_general(xg, w3, (((1,), (1,)), ((), ())),
                            preferred_element_type=jnp.float32)
        h = (a * (1.0 / (1.0 + jnp.exp(-a))) * b).astype(jnp.bfloat16)
        part = lax.dot_general(h, w2, (((1,), (1,)), ((), ())),
                               preferred_element_type=jnp.float32)

        @pl.when(j == 0)
        def _():
            acc_ref[sl, :] = part

        @pl.when((j > 0) & (j < nj - 1))
        def _():
            acc_ref[sl, :] += part

        @pl.when(j == nj - 1)
        def _():
            acc_ref[sl, :] = ((acc_ref[sl, :] + part)
                              * sw_ref[sl, 0:1].astype(jnp.float32))

        return carry

    lax.fori_loop(0, nb, compute_mb, 0)

    @pl.when(j == nj - 1)
    def _():
        def fire(mb, carry):
            r0 = (pb + mb) * _ROW_BLK
            pltpu.make_async_copy(
                acc_ref.at[pl.ds(r0, _ROW_BLK)],
                ys_hbm.at[pl.ds(r0, _ROW_BLK)], sem_o).start()
            return carry

        lax.fori_loop(0, nb, fire, 0)

        def drain(mb, carry):
            pltpu.make_async_copy(
                acc_ref.at[pl.ds(0, _ROW_BLK)],
                ys_hbm.at[pl.ds(0, _ROW_BLK)], sem_o).wait()
            return carry

        lax.fori_loop(0, nb, drain, 0)


def _run_gmm(pref, xg, w1, w3, w2, sw2d):
    S_max, D = xg.shape
    FFN = w1.shape[1]
    J = FFN // _FFN_BLK
    grid_spec = pltpu.PrefetchScalarGridSpec(
        num_scalar_prefetch=1,
        grid=(J, _E),
        in_specs=[
            pl.BlockSpec(memory_space=pl.ANY),
            pl.BlockSpec((1, _FFN_BLK, D), lambda j, e, pref: (e, j, 0)),
            pl.BlockSpec((1, _FFN_BLK, D), lambda j, e, pref: (e, j, 0)),
            pl.BlockSpec((1, D, _FFN_BLK), lambda j, e, pref: (e, 0, j)),
            pl.BlockSpec((S_max, _E), lambda j, e, pref: (0, 0)),
        ],
        out_specs=pl.BlockSpec(memory_space=pl.ANY),
        scratch_shapes=[
            pltpu.VMEM((S_max, D), jnp.float32),
            pltpu.VMEM((S_max, D), jnp.bfloat16),
            pltpu.SemaphoreType.DMA,
            pltpu.SemaphoreType.DMA,
        ],
    )
    return pl.pallas_call(
        _gmm_body,
        grid_spec=grid_spec,
        out_shape=jax.ShapeDtypeStruct((S_max, D), jnp.float32),
        compiler_params=pltpu.CompilerParams(
            dimension_semantics=("arbitrary", "arbitrary")),
    )(pref, xg, w1, w3, w2, sw2d)


def _run_sc_combine(ys, inv0, inv1):
    T = inv0.shape[0]
    D = ys.shape[1]
    info = plsc.get_sparse_core_info()
    nw = info.num_cores * info.num_subcores
    tok_pw = T // nw
    ch = 32
    n_ch = tok_pw // ch
    vecs_per_row = D // 16
    mesh = plsc.VectorSubcoreMesh(core_axis_name="c", subcore_axis_name="s")

    @functools.partial(
        pl.kernel,
        out_type=jax.ShapeDtypeStruct((T, D), jnp.float32),
        mesh=mesh,
        scratch_types=[
            pltpu.VMEM((ch,), jnp.int32),
            pltpu.VMEM((ch,), jnp.int32),
            pltpu.VMEM((ch, D), jnp.float32),
            pltpu.VMEM((ch, D), jnp.float32),
            pltpu.SemaphoreType.DMA,
            pltpu.SemaphoreType.DMA,
        ],
    )
    def k(ys_hbm, i0_hbm, i1_hbm, out_hbm, i0_v, i1_v, r0, r1, s0, s1):
        wid = lax.axis_index("s") * info.num_cores + lax.axis_index("c")
        base = wid * tok_pw
        for c in range(n_ch):
            off = base + c * ch
            pltpu.sync_copy(i0_hbm.at[pl.ds(off, ch)], i0_v)
            pltpu.sync_copy(i1_hbm.at[pl.ds(off, ch)], i1_v)
            cp0 = pltpu.async_copy(ys_hbm.at[i0_v], r0, s0)
            cp1 = pltpu.async_copy(ys_hbm.at[i1_v], r1, s1)
            cp0.wait()
            cp1.wait()

            def row_add(r, carry):
                for v in range(vecs_per_row):
                    sl = pl.ds(v * 16, 16)
                    r0[r, sl] = r0[r, sl] + r1[r, sl]
                return carry

            lax.fori_loop(0, ch, row_add, 0)
            pltpu.sync_copy(r0, out_hbm.at[pl.ds(off, ch)])

    return k(ys, inv0, inv1)


def kernel(hidden_states, gate_w, w1, w3, w2):
    B, S, D = hidden_states.shape
    T = B * S
    x = hidden_states.reshape(T, D)
    S_max = 2 * T + _E * _ROW_BLK

    wpad, ipad = _run_router(x, gate_w)
    tw = wpad[:, :2]
    ti = ipad[:, :2]

    e_flat = jnp.concatenate([ti[:, 0], ti[:, 1]])
    w_flat = jnp.concatenate([tw[:, 0], tw[:, 1]])
    onehot = (e_flat[:, None] == jnp.arange(_E, dtype=jnp.int32)[None, :]
              ).astype(jnp.int32)
    ranks_all = jnp.cumsum(onehot, axis=0) - onehot
    rank = jnp.take_along_axis(ranks_all, e_flat[:, None], axis=1)[:, 0]
    counts = jnp.sum(onehot, axis=0)
    pcounts = ((counts + _ROW_BLK - 1) // _ROW_BLK) * _ROW_BLK
    pstart = jnp.concatenate(
        [jnp.zeros((1,), jnp.int32), jnp.cumsum(pcounts)[:-1].astype(jnp.int32)])
    pos = pstart[e_flat] + rank

    tok2 = jnp.arange(2 * T, dtype=jnp.int32) % T
    tok2d = jnp.zeros((S_max, _E), jnp.int32).at[pos].set(
        jnp.broadcast_to(tok2[:, None], (2 * T, _E)))
    sw2d = jnp.zeros((S_max, _E), jnp.bfloat16).at[pos].set(
        jnp.broadcast_to(w_flat[:, None].astype(jnp.bfloat16), (2 * T, _E)))

    pref = jnp.stack([pcounts // _ROW_BLK, pstart // _ROW_BLK]
                     ).astype(jnp.int32)
    inv0 = pos[:T]
    inv1 = pos[T:]

    xg = _run_gather_tc(x, tok2d, S_max)
    ys = _run_gmm(pref, xg, w1, w3, w2, sw2d)
    out = _run_sc_combine(ys, inv0, inv1)
    return out.reshape(B, S, D)
